# baseline (speedup 1.0000x reference)
import functools

import jax
import jax.numpy as jnp
from jax import lax
from jax.experimental import pallas as pl
from jax.experimental.pallas import tpu as pltpu
from jax.experimental.pallas import tpu_sc as plsc

N = 10000
E = 320000
NC = 2
NS = 16
NW = NC * NS
EPW = E // NW
C = 125
NCH = EPW // C
NP = 10240
RPT = NP // NS
ZR = 128
EPT = E // NS
NCH2 = EPT // C

_MESH = plsc.VectorSubcoreMesh(core_axis_name="c", subcore_axis_name="s")
_SC_PARAMS = pltpu.CompilerParams(needs_layout_passes=False, use_tc_tiling_on_sc=False)


@functools.partial(
    pl.kernel,
    mesh=_MESH,
    compiler_params=_SC_PARAMS,
    out_type=jax.ShapeDtypeStruct((NW * 2 * N,), jnp.float32),
    scratch_types=[
        pltpu.VMEM((EPW,), jnp.int32),
        pltpu.VMEM((N,), jnp.float32),
        pltpu.VMEM((N,), jnp.float32),
    ],
)
def _deg_kernel(src_hbm, dst_hbm, out_hbm, idx_v, dego_v, degi_v):
    cid = lax.axis_index("c")
    sid = lax.axis_index("s")
    wid = cid * NS + sid

    zero16 = jnp.zeros((16,), jnp.float32)

    def zloop(i, carry):
        dego_v[pl.ds(i * 16, 16)] = zero16
        degi_v[pl.ds(i * 16, 16)] = zero16
        return carry

    lax.fori_loop(0, N // 16, zloop, 0)

    ones16 = jnp.ones((16,), jnp.float32)

    pltpu.sync_copy(src_hbm.at[pl.ds(wid * EPW, EPW)], idx_v)

    def sloop(i, carry):
        s = idx_v[pl.ds(i * 16, 16)]
        plsc.addupdate_scatter(dego_v, [s], ones16)
        return carry

    lax.fori_loop(0, EPW // 16, sloop, 0)

    pltpu.sync_copy(dst_hbm.at[pl.ds(wid * EPW, EPW)], idx_v)

    def dloop(i, carry):
        t = idx_v[pl.ds(i * 16, 16)]
        plsc.addupdate_scatter(degi_v, [t], ones16)
        return carry

    lax.fori_loop(0, EPW // 16, dloop, 0)

    pltpu.sync_copy(dego_v, out_hbm.at[pl.ds(wid * 2 * N, N)])
    pltpu.sync_copy(degi_v, out_hbm.at[pl.ds(wid * 2 * N + N, N)])


def _make_agg_fsplit(H):
    @functools.partial(
        pl.kernel,
        mesh=_MESH,
        compiler_params=_SC_PARAMS,
        out_type=jax.ShapeDtypeStruct((NC, NP, H), jnp.float32),
        scratch_types=[
            pltpu.VMEM((NCH2, C), jnp.int32),
            pltpu.VMEM((NCH2, C), jnp.int32),
            pltpu.VMEM((C, H), jnp.float32),
            pltpu.VMEM((C, H), jnp.float32),
            pltpu.VMEM((C, H), jnp.float32),
            pltpu.VMEM((C, H), jnp.float32),
            pltpu.VMEM((ZR, H), jnp.float32),
            pltpu.VMEM_SHARED((NP, H), jnp.float32),
            pltpu.SemaphoreType.DMA,
            pltpu.SemaphoreType.DMA,
            pltpu.SemaphoreType.DMA,
            pltpu.SemaphoreType.DMA,
        ],
    )
    def agg(m_hbm, src_hbm, dst_hbm, out_hbm, srcv, dstv,
            r0, r1, r2, r3, zbuf, acc, s0, s1, s2, s3):
        cid = lax.axis_index("c")
        sid = lax.axis_index("s")
        bufs = (r0, r1, r2, r3)
        sems = (s0, s1, s2, s3)
        NB = 4
        vpr = H // 16

        pltpu.sync_copy(src_hbm.at[sid], srcv)
        pltpu.sync_copy(dst_hbm.at[sid], dstv)
        m_c = m_hbm.at[cid]

        for b in range(NB):
            pltpu.async_copy(m_c.at[srcv.at[b]], bufs[b], sems[b])

        zero16 = jnp.zeros((16,), jnp.float32)

        def zb(i, carry):
            zbuf[i // vpr, pl.ds((i % vpr) * 16, 16)] = zero16
            return carry

        lax.fori_loop(0, ZR * vpr, zb, 0)

        for j in range(RPT // ZR):
            pltpu.sync_copy(zbuf, acc.at[pl.ds(sid * RPT + j * ZR, ZR), :])
        plsc.subcore_barrier()

        def chunk(j, carry):
            for b in range(NB):
                q = j * NB + b
                pltpu.make_async_copy(m_c.at[srcv.at[q]], bufs[b], sems[b]).wait()
                pltpu.sync_copy(bufs[b], acc.at[dstv.at[q]], add=True)

                @pl.when(q + NB < NCH2)
                def _():
                    pltpu.async_copy(m_c.at[srcv.at[q + NB]], bufs[b], sems[b])

            return carry

        lax.fori_loop(0, NCH2 // NB, chunk, 0)
        plsc.subcore_barrier()

        pltpu.sync_copy(
            acc.at[pl.ds(sid * RPT, RPT), :],
            out_hbm.at[cid, pl.ds(sid * RPT, RPT), :],
        )

    return agg


def _make_agg(F):
    @functools.partial(
        pl.kernel,
        mesh=_MESH,
        compiler_params=_SC_PARAMS,
        out_type=jax.ShapeDtypeStruct((NC, NP, F), jnp.float32),
        scratch_types=[
            pltpu.VMEM((NCH, C), jnp.int32),
            pltpu.VMEM((NCH, C), jnp.int32),
            pltpu.VMEM((C, F), jnp.float32),
            pltpu.VMEM((C, F), jnp.float32),
            pltpu.VMEM((C, F), jnp.float32),
            pltpu.VMEM((C, F), jnp.float32),
            pltpu.VMEM((ZR, F), jnp.float32),
            pltpu.VMEM_SHARED((NP, F), jnp.float32),
            pltpu.SemaphoreType.DMA,
            pltpu.SemaphoreType.DMA,
            pltpu.SemaphoreType.DMA,
            pltpu.SemaphoreType.DMA,
        ],
    )
    def agg(m_hbm, src_hbm, dst_hbm, out_hbm, srcv, dstv,
            r0, r1, r2, r3, zbuf, acc, s0, s1, s2, s3):
        cid = lax.axis_index("c")
        sid = lax.axis_index("s")
        wid = cid * NS + sid
        bufs = (r0, r1, r2, r3)
        sems = (s0, s1, s2, s3)
        NB = 4

        pltpu.sync_copy(src_hbm.at[wid], srcv)
        pltpu.sync_copy(dst_hbm.at[wid], dstv)

        for b in range(NB):
            pltpu.async_copy(m_hbm.at[srcv.at[b]], bufs[b], sems[b])

        zero16 = jnp.zeros((16,), jnp.float32)
        vpr = F // 16

        def zb(i, carry):
            zbuf[i // vpr, pl.ds((i % vpr) * 16, 16)] = zero16
            return carry

        lax.fori_loop(0, ZR * vpr, zb, 0)

        for j in range(RPT // ZR):
            pltpu.sync_copy(zbuf, acc.at[pl.ds(sid * RPT + j * ZR, ZR), :])
        plsc.subcore_barrier()

        def chunk(j, carry):
            for b in range(NB):
                q = j * NB + b
                pltpu.make_async_copy(m_hbm.at[srcv.at[q]], bufs[b], sems[b]).wait()
                pltpu.sync_copy(bufs[b], acc.at[dstv.at[q]], add=True)

                @pl.when(q + NB < NCH)
                def _():
                    pltpu.async_copy(m_hbm.at[srcv.at[q + NB]], bufs[b], sems[b])

            return carry

        lax.fori_loop(0, NCH // NB, chunk, 0)
        plsc.subcore_barrier()

        pltpu.sync_copy(
            acc.at[pl.ds(sid * RPT, RPT), :],
            out_hbm.at[cid, pl.ds(sid * RPT, RPT), :],
        )

    return agg


_agg128 = _make_agg_fsplit(64)
_agg32x2 = _make_agg_fsplit(32)
_agg16 = _make_agg(16)


_B = 2000


def _tc0_body(degp_ref, x_ref, m1_ref, no_ref, ni_ref):
    deg = jnp.sum(degp_ref[...], axis=0)
    norm = jnp.where(deg > 0, lax.rsqrt(deg), 0.0)
    nT = jnp.transpose(norm)
    no = jnp.broadcast_to(nT[:, 0:1], (N, 128))
    ni = jnp.broadcast_to(nT[:, 1:2], (N, 128))
    no_ref[...] = no
    ni_ref[...] = ni
    m1 = x_ref[...] * no
    m1_ref[0] = m1[:, :64]
    m1_ref[1] = m1[:, 64:]


def _tc0(degp, x):
    return pl.pallas_call(
        _tc0_body,
        grid=(1,),
        in_specs=[
            pl.BlockSpec((NW, 2, N), lambda i: (0, 0, 0)),
            pl.BlockSpec((N, 128), lambda i: (0, 0)),
        ],
        out_specs=[
            pl.BlockSpec((2, N, 64), lambda i: (0, 0, 0)),
            pl.BlockSpec((N, 128), lambda i: (0, 0)),
            pl.BlockSpec((N, 128), lambda i: (0, 0)),
        ],
        out_shape=[
            jax.ShapeDtypeStruct((2, N, 64), jnp.float32),
            jax.ShapeDtypeStruct((N, 128), jnp.float32),
            jax.ShapeDtypeStruct((N, 128), jnp.float32),
        ],
    )(degp, x)


def _tc1_body(p_ref, ni_ref, no_ref, W1_ref, b1_ref, W2_ref, m2_ref):
    agg = jnp.concatenate([p_ref[0], p_ref[1]], axis=1) * ni_ref[...]
    h1 = jnp.maximum(
        jnp.dot(agg, W1_ref[...], preferred_element_type=jnp.float32)
        + b1_ref[...],
        0.0,
    )
    m2 = (
        jnp.dot(h1, W2_ref[...], preferred_element_type=jnp.float32)
        * no_ref[...][:, :64]
    )
    m2_ref[0] = m2[:, :32]
    m2_ref[1] = m2[:, 32:]


def _tc1(p1, ni, no, W1, b1, W2):
    return pl.pallas_call(
        _tc1_body,
        grid=(N // _B,),
        in_specs=[
            pl.BlockSpec((2, _B, 64), lambda i: (0, i, 0)),
            pl.BlockSpec((_B, 128), lambda i: (i, 0)),
            pl.BlockSpec((_B, 128), lambda i: (i, 0)),
            pl.BlockSpec((128, 128), lambda i: (0, 0)),
            pl.BlockSpec((1, 128), lambda i: (0, 0)),
            pl.BlockSpec((128, 64), lambda i: (0, 0)),
        ],
        out_specs=pl.BlockSpec((2, _B, 32), lambda i: (0, i, 0)),
        out_shape=jax.ShapeDtypeStruct((2, N, 32), jnp.float32),
    )(p1, ni, no, W1, b1, W2)


def _tc2_body(p_ref, ni_ref, no_ref, b2_ref, W3_ref, m3_ref):
    h2 = jnp.maximum(
        jnp.concatenate([p_ref[0], p_ref[1]], axis=1) * ni_ref[...][:, :64]
        + b2_ref[...],
        0.0,
    )
    m3_ref[...] = (
        jnp.dot(h2, W3_ref[...], preferred_element_type=jnp.float32)
        * no_ref[...][:, :16]
    )


def _tc2(p2, ni, no, b2, W3):
    return pl.pallas_call(
        _tc2_body,
        grid=(N // _B,),
        in_specs=[
            pl.BlockSpec((2, _B, 32), lambda i: (0, i, 0)),
            pl.BlockSpec((_B, 128), lambda i: (i, 0)),
            pl.BlockSpec((_B, 128), lambda i: (i, 0)),
            pl.BlockSpec((1, 64), lambda i: (0, 0)),
            pl.BlockSpec((64, 16), lambda i: (0, 0)),
        ],
        out_specs=pl.BlockSpec((_B, 16), lambda i: (i, 0)),
        out_shape=jax.ShapeDtypeStruct((N, 16), jnp.float32),
    )(p2, ni, no, b2, W3)


def _tc3_body(p_ref, ni_ref, b3_ref, h3_ref):
    h3_ref[...] = (p_ref[0] + p_ref[1]) * ni_ref[...][:, :16] + b3_ref[...]


def _tc3(p3, ni, b3):
    return pl.pallas_call(
        _tc3_body,
        grid=(N // _B,),
        in_specs=[
            pl.BlockSpec((2, _B, 16), lambda i: (0, i, 0)),
            pl.BlockSpec((_B, 128), lambda i: (i, 0)),
            pl.BlockSpec((1, 16), lambda i: (0, 0)),
        ],
        out_specs=pl.BlockSpec((_B, 16), lambda i: (i, 0)),
        out_shape=jax.ShapeDtypeStruct((N, 16), jnp.float32),
    )(p3, ni, b3)


def kernel(x, edge_index, W1, b1, W2, b2, W3, b3):
    src = edge_index[0].astype(jnp.int32)
    dst = edge_index[1].astype(jnp.int32)
    src_w = src.reshape(NW, NCH, C)
    dst_w = dst.reshape(NW, NCH, C)
    src_t = src.reshape(NS, NCH2, C)
    dst_t = dst.reshape(NS, NCH2, C)

    degp = _deg_kernel(src, dst)
    m1, no, ni = _tc0(degp.reshape(NW, 2, N), x)

    p1 = _agg128(m1, src_t, dst_t)
    m2 = _tc1(p1, ni, no, W1, b1.reshape(1, 128), W2)

    p2 = _agg32x2(m2, src_t, dst_t)
    m3 = _tc2(p2, ni, no, b2.reshape(1, 64), W3)

    p3 = _agg16(m3, src_w, dst_w)
    return _tc3(p3, ni, b3.reshape(1, 16))


# speedup vs baseline: 1.0289x; 1.0289x over previous
import functools

import jax
import jax.numpy as jnp
from jax import lax
from jax.experimental import pallas as pl
from jax.experimental.pallas import tpu as pltpu
from jax.experimental.pallas import tpu_sc as plsc

N = 10000
E = 320000
NC = 2
NS = 16
NW = NC * NS
EPW = E // NW
C = 125
NCH = EPW // C
NP = 10240
RPT = NP // NS
ZR = 128
EPT = E // NS
NCH2 = EPT // C

_MESH = plsc.VectorSubcoreMesh(core_axis_name="c", subcore_axis_name="s")
_SC_PARAMS = pltpu.CompilerParams(
    needs_layout_passes=False,
    use_tc_tiling_on_sc=False,
    skip_device_barrier=True,
)


@functools.partial(
    pl.kernel,
    mesh=_MESH,
    compiler_params=_SC_PARAMS,
    out_type=jax.ShapeDtypeStruct((NW * 2 * N,), jnp.float32),
    scratch_types=[
        pltpu.VMEM((EPW,), jnp.int32),
        pltpu.VMEM((N,), jnp.float32),
        pltpu.VMEM((N,), jnp.float32),
    ],
)
def _deg_kernel(src_hbm, dst_hbm, out_hbm, idx_v, dego_v, degi_v):
    cid = lax.axis_index("c")
    sid = lax.axis_index("s")
    wid = cid * NS + sid

    zero16 = jnp.zeros((16,), jnp.float32)

    def zloop(i, carry):
        dego_v[pl.ds(i * 16, 16)] = zero16
        degi_v[pl.ds(i * 16, 16)] = zero16
        return carry

    lax.fori_loop(0, N // 16, zloop, 0)

    ones16 = jnp.ones((16,), jnp.float32)

    pltpu.sync_copy(src_hbm.at[pl.ds(wid * EPW, EPW)], idx_v)

    def sloop(i, carry):
        s = idx_v[pl.ds(i * 16, 16)]
        plsc.addupdate_scatter(dego_v, [s], ones16)
        return carry

    lax.fori_loop(0, EPW // 16, sloop, 0)

    pltpu.sync_copy(dst_hbm.at[pl.ds(wid * EPW, EPW)], idx_v)

    def dloop(i, carry):
        t = idx_v[pl.ds(i * 16, 16)]
        plsc.addupdate_scatter(degi_v, [t], ones16)
        return carry

    lax.fori_loop(0, EPW // 16, dloop, 0)

    pltpu.sync_copy(dego_v, out_hbm.at[pl.ds(wid * 2 * N, N)])
    pltpu.sync_copy(degi_v, out_hbm.at[pl.ds(wid * 2 * N + N, N)])


def _make_agg_fsplit(H):
    @functools.partial(
        pl.kernel,
        mesh=_MESH,
        compiler_params=_SC_PARAMS,
        out_type=jax.ShapeDtypeStruct((NC, NP, H), jnp.float32),
        scratch_types=[
            pltpu.VMEM((NCH2, C), jnp.int32),
            pltpu.VMEM((NCH2, C), jnp.int32),
            pltpu.VMEM((C, H), jnp.float32),
            pltpu.VMEM((C, H), jnp.float32),
            pltpu.VMEM((C, H), jnp.float32),
            pltpu.VMEM((C, H), jnp.float32),
            pltpu.VMEM((ZR, H), jnp.float32),
            pltpu.VMEM_SHARED((NP, H), jnp.float32),
            pltpu.SemaphoreType.DMA,
            pltpu.SemaphoreType.DMA,
            pltpu.SemaphoreType.DMA,
            pltpu.SemaphoreType.DMA,
        ],
    )
    def agg(m_hbm, src_hbm, dst_hbm, out_hbm, srcv, dstv,
            r0, r1, r2, r3, zbuf, acc, s0, s1, s2, s3):
        cid = lax.axis_index("c")
        sid = lax.axis_index("s")
        bufs = (r0, r1, r2, r3)
        sems = (s0, s1, s2, s3)
        NB = 4
        vpr = H // 16

        pltpu.sync_copy(src_hbm.at[sid], srcv)
        pltpu.sync_copy(dst_hbm.at[sid], dstv)
        m_c = m_hbm.at[cid]

        for b in range(NB):
            pltpu.async_copy(m_c.at[srcv.at[b]], bufs[b], sems[b])

        zero16 = jnp.zeros((16,), jnp.float32)

        def zb(i, carry):
            zbuf[i // vpr, pl.ds((i % vpr) * 16, 16)] = zero16
            return carry

        lax.fori_loop(0, ZR * vpr, zb, 0)

        for j in range(RPT // ZR):
            pltpu.sync_copy(zbuf, acc.at[pl.ds(sid * RPT + j * ZR, ZR), :])
        plsc.subcore_barrier()

        def chunk(j, carry):
            for b in range(NB):
                q = j * NB + b
                pltpu.make_async_copy(m_c.at[srcv.at[q]], bufs[b], sems[b]).wait()
                pltpu.sync_copy(bufs[b], acc.at[dstv.at[q]], add=True)

                @pl.when(q + NB < NCH2)
                def _():
                    pltpu.async_copy(m_c.at[srcv.at[q + NB]], bufs[b], sems[b])

            return carry

        lax.fori_loop(0, NCH2 // NB, chunk, 0)
        plsc.subcore_barrier()

        pltpu.sync_copy(
            acc.at[pl.ds(sid * RPT, RPT), :],
            out_hbm.at[cid, pl.ds(sid * RPT, RPT), :],
        )

    return agg


def _make_agg(F):
    @functools.partial(
        pl.kernel,
        mesh=_MESH,
        compiler_params=_SC_PARAMS,
        out_type=jax.ShapeDtypeStruct((NC, NP, F), jnp.float32),
        scratch_types=[
            pltpu.VMEM((NCH, C), jnp.int32),
            pltpu.VMEM((NCH, C), jnp.int32),
            pltpu.VMEM((C, F), jnp.float32),
            pltpu.VMEM((C, F), jnp.float32),
            pltpu.VMEM((C, F), jnp.float32),
            pltpu.VMEM((C, F), jnp.float32),
            pltpu.VMEM((ZR, F), jnp.float32),
            pltpu.VMEM_SHARED((NP, F), jnp.float32),
            pltpu.SemaphoreType.DMA,
            pltpu.SemaphoreType.DMA,
            pltpu.SemaphoreType.DMA,
            pltpu.SemaphoreType.DMA,
        ],
    )
    def agg(m_hbm, src_hbm, dst_hbm, out_hbm, srcv, dstv,
            r0, r1, r2, r3, zbuf, acc, s0, s1, s2, s3):
        cid = lax.axis_index("c")
        sid = lax.axis_index("s")
        wid = cid * NS + sid
        bufs = (r0, r1, r2, r3)
        sems = (s0, s1, s2, s3)
        NB = 4

        pltpu.sync_copy(src_hbm.at[wid], srcv)
        pltpu.sync_copy(dst_hbm.at[wid], dstv)

        for b in range(NB):
            pltpu.async_copy(m_hbm.at[srcv.at[b]], bufs[b], sems[b])

        zero16 = jnp.zeros((16,), jnp.float32)
        vpr = F // 16

        def zb(i, carry):
            zbuf[i // vpr, pl.ds((i % vpr) * 16, 16)] = zero16
            return carry

        lax.fori_loop(0, ZR * vpr, zb, 0)

        for j in range(RPT // ZR):
            pltpu.sync_copy(zbuf, acc.at[pl.ds(sid * RPT + j * ZR, ZR), :])
        plsc.subcore_barrier()

        def chunk(j, carry):
            for b in range(NB):
                q = j * NB + b
                pltpu.make_async_copy(m_hbm.at[srcv.at[q]], bufs[b], sems[b]).wait()
                pltpu.sync_copy(bufs[b], acc.at[dstv.at[q]], add=True)

                @pl.when(q + NB < NCH)
                def _():
                    pltpu.async_copy(m_hbm.at[srcv.at[q + NB]], bufs[b], sems[b])

            return carry

        lax.fori_loop(0, NCH // NB, chunk, 0)
        plsc.subcore_barrier()

        pltpu.sync_copy(
            acc.at[pl.ds(sid * RPT, RPT), :],
            out_hbm.at[cid, pl.ds(sid * RPT, RPT), :],
        )

    return agg


_agg128 = _make_agg_fsplit(64)
_agg64 = _make_agg(64)
_agg16 = _make_agg(16)


_B = 2000


def _tc0_body(degp_ref, x_ref, m1_ref, no_ref, ni_ref):
    deg = jnp.sum(degp_ref[...], axis=0)
    norm = jnp.where(deg > 0, lax.rsqrt(deg), 0.0)
    nT = jnp.transpose(norm)
    no = jnp.broadcast_to(nT[:, 0:1], (N, 128))
    ni = jnp.broadcast_to(nT[:, 1:2], (N, 128))
    no_ref[...] = no
    ni_ref[...] = ni
    m1 = x_ref[...] * no
    m1_ref[0] = m1[:, :64]
    m1_ref[1] = m1[:, 64:]


def _tc0(degp, x):
    return pl.pallas_call(
        _tc0_body,
        grid=(1,),
        in_specs=[
            pl.BlockSpec((NW, 2, N), lambda i: (0, 0, 0)),
            pl.BlockSpec((N, 128), lambda i: (0, 0)),
        ],
        out_specs=[
            pl.BlockSpec((2, N, 64), lambda i: (0, 0, 0)),
            pl.BlockSpec((N, 128), lambda i: (0, 0)),
            pl.BlockSpec((N, 128), lambda i: (0, 0)),
        ],
        out_shape=[
            jax.ShapeDtypeStruct((2, N, 64), jnp.float32),
            jax.ShapeDtypeStruct((N, 128), jnp.float32),
            jax.ShapeDtypeStruct((N, 128), jnp.float32),
        ],
    )(degp, x)


def _tc1_body(p_ref, ni_ref, no_ref, W1_ref, b1_ref, W2_ref, m2_ref):
    agg = jnp.concatenate([p_ref[0], p_ref[1]], axis=1) * ni_ref[...]
    h1 = jnp.maximum(
        jnp.dot(agg, W1_ref[...], preferred_element_type=jnp.float32)
        + b1_ref[...],
        0.0,
    )
    m2_ref[...] = (
        jnp.dot(h1, W2_ref[...], preferred_element_type=jnp.float32)
        * no_ref[...][:, :64]
    )


def _tc1(p1, ni, no, W1, b1, W2):
    return pl.pallas_call(
        _tc1_body,
        grid=(N // _B,),
        in_specs=[
            pl.BlockSpec((2, _B, 64), lambda i: (0, i, 0)),
            pl.BlockSpec((_B, 128), lambda i: (i, 0)),
            pl.BlockSpec((_B, 128), lambda i: (i, 0)),
            pl.BlockSpec((128, 128), lambda i: (0, 0)),
            pl.BlockSpec((1, 128), lambda i: (0, 0)),
            pl.BlockSpec((128, 64), lambda i: (0, 0)),
        ],
        out_specs=pl.BlockSpec((_B, 64), lambda i: (i, 0)),
        out_shape=jax.ShapeDtypeStruct((N, 64), jnp.float32),
    )(p1, ni, no, W1, b1, W2)


def _tc2_body(p_ref, ni_ref, no_ref, b2_ref, W3_ref, m3_ref):
    h2 = jnp.maximum(
        (p_ref[0] + p_ref[1]) * ni_ref[...][:, :64] + b2_ref[...], 0.0
    )
    m3_ref[...] = (
        jnp.dot(h2, W3_ref[...], preferred_element_type=jnp.float32)
        * no_ref[...][:, :16]
    )


def _tc2(p2, ni, no, b2, W3):
    return pl.pallas_call(
        _tc2_body,
        grid=(N // _B,),
        in_specs=[
            pl.BlockSpec((2, _B, 64), lambda i: (0, i, 0)),
            pl.BlockSpec((_B, 128), lambda i: (i, 0)),
            pl.BlockSpec((_B, 128), lambda i: (i, 0)),
            pl.BlockSpec((1, 64), lambda i: (0, 0)),
            pl.BlockSpec((64, 16), lambda i: (0, 0)),
        ],
        out_specs=pl.BlockSpec((_B, 16), lambda i: (i, 0)),
        out_shape=jax.ShapeDtypeStruct((N, 16), jnp.float32),
    )(p2, ni, no, b2, W3)


def _tc3_body(p_ref, ni_ref, b3_ref, h3_ref):
    h3_ref[...] = (p_ref[0] + p_ref[1]) * ni_ref[...][:, :16] + b3_ref[...]


def _tc3(p3, ni, b3):
    return pl.pallas_call(
        _tc3_body,
        grid=(N // _B,),
        in_specs=[
            pl.BlockSpec((2, _B, 16), lambda i: (0, i, 0)),
            pl.BlockSpec((_B, 128), lambda i: (i, 0)),
            pl.BlockSpec((1, 16), lambda i: (0, 0)),
        ],
        out_specs=pl.BlockSpec((_B, 16), lambda i: (i, 0)),
        out_shape=jax.ShapeDtypeStruct((N, 16), jnp.float32),
    )(p3, ni, b3)


def kernel(x, edge_index, W1, b1, W2, b2, W3, b3):
    src = edge_index[0].astype(jnp.int32)
    dst = edge_index[1].astype(jnp.int32)
    src_w = src.reshape(NW, NCH, C)
    dst_w = dst.reshape(NW, NCH, C)
    src_t = src.reshape(NS, NCH2, C)
    dst_t = dst.reshape(NS, NCH2, C)

    degp = _deg_kernel(src, dst)
    m1, no, ni = _tc0(degp.reshape(NW, 2, N), x)

    p1 = _agg128(m1, src_t, dst_t)
    m2 = _tc1(p1, ni, no, W1, b1.reshape(1, 128), W2)

    p2 = _agg64(m2, src_w, dst_w)
    m3 = _tc2(p2, ni, no, b2.reshape(1, 64), W3)

    p3 = _agg16(m3, src_w, dst_w)
    return _tc3(p3, ni, b3.reshape(1, 16))


# speedup vs baseline: 1.1377x; 1.1057x over previous
import functools

import jax
import jax.numpy as jnp
from jax import lax
from jax.experimental import pallas as pl
from jax.experimental.pallas import tpu as pltpu
from jax.experimental.pallas import tpu_sc as plsc

N = 10000
E = 320000
NC = 2
NS = 16
NW = NC * NS
EPW = E // NW
C = 125
NCH = EPW // C
NP = 10240
RPT = NP // NS
ZR = 128
EPT = E // NS
NCH2 = EPT // C

_MESH = plsc.VectorSubcoreMesh(core_axis_name="c", subcore_axis_name="s")
_SC_PARAMS = pltpu.CompilerParams(
    needs_layout_passes=False,
    use_tc_tiling_on_sc=False,
    skip_device_barrier=True,
)


@functools.partial(
    pl.kernel,
    mesh=_MESH,
    compiler_params=_SC_PARAMS,
    out_type=jax.ShapeDtypeStruct((NW * 2 * N,), jnp.float32),
    scratch_types=[
        pltpu.VMEM((EPW,), jnp.int32),
        pltpu.VMEM((N,), jnp.float32),
        pltpu.VMEM((N,), jnp.float32),
    ],
)
def _deg_kernel(src_hbm, dst_hbm, out_hbm, idx_v, dego_v, degi_v):
    cid = lax.axis_index("c")
    sid = lax.axis_index("s")
    wid = cid * NS + sid

    zero16 = jnp.zeros((16,), jnp.float32)

    def zloop(i, carry):
        dego_v[pl.ds(i * 16, 16)] = zero16
        degi_v[pl.ds(i * 16, 16)] = zero16
        return carry

    lax.fori_loop(0, N // 16, zloop, 0)

    ones16 = jnp.ones((16,), jnp.float32)

    pltpu.sync_copy(src_hbm.at[pl.ds(wid * EPW, EPW)], idx_v)

    def sloop(i, carry):
        s = idx_v[pl.ds(i * 16, 16)]
        plsc.addupdate_scatter(dego_v, [s], ones16)
        return carry

    lax.fori_loop(0, EPW // 16, sloop, 0)

    pltpu.sync_copy(dst_hbm.at[pl.ds(wid * EPW, EPW)], idx_v)

    def dloop(i, carry):
        t = idx_v[pl.ds(i * 16, 16)]
        plsc.addupdate_scatter(degi_v, [t], ones16)
        return carry

    lax.fori_loop(0, EPW // 16, dloop, 0)

    pltpu.sync_copy(dego_v, out_hbm.at[pl.ds(wid * 2 * N, N)])
    pltpu.sync_copy(degi_v, out_hbm.at[pl.ds(wid * 2 * N + N, N)])


def _make_agg_fsplit(H, dt=jnp.float32):
    @functools.partial(
        pl.kernel,
        mesh=_MESH,
        compiler_params=_SC_PARAMS,
        out_type=jax.ShapeDtypeStruct((NC, NP, H), dt),
        scratch_types=[
            pltpu.VMEM((NCH2, C), jnp.int32),
            pltpu.VMEM((NCH2, C), jnp.int32),
            pltpu.VMEM((C, H), dt),
            pltpu.VMEM((C, H), dt),
            pltpu.VMEM((C, H), dt),
            pltpu.VMEM((C, H), dt),
            pltpu.VMEM((ZR, H), dt),
            pltpu.VMEM_SHARED((NP, H), dt),
            pltpu.SemaphoreType.DMA,
            pltpu.SemaphoreType.DMA,
            pltpu.SemaphoreType.DMA,
            pltpu.SemaphoreType.DMA,
        ],
    )
    def agg(m_hbm, src_hbm, dst_hbm, out_hbm, srcv, dstv,
            r0, r1, r2, r3, zbuf, acc, s0, s1, s2, s3):
        cid = lax.axis_index("c")
        sid = lax.axis_index("s")
        bufs = (r0, r1, r2, r3)
        sems = (s0, s1, s2, s3)
        NB = 4
        lanes = 16 if dt == jnp.float32 else 32
        vpr = H // lanes

        pltpu.sync_copy(src_hbm.at[sid], srcv)
        pltpu.sync_copy(dst_hbm.at[sid], dstv)
        m_c = m_hbm.at[cid]

        for b in range(NB):
            pltpu.async_copy(m_c.at[srcv.at[b]], bufs[b], sems[b])

        zerov = jnp.zeros((lanes,), dt)

        def zb(i, carry):
            zbuf[i // vpr, pl.ds((i % vpr) * lanes, lanes)] = zerov
            return carry

        lax.fori_loop(0, ZR * vpr, zb, 0)

        for j in range(RPT // ZR):
            pltpu.sync_copy(zbuf, acc.at[pl.ds(sid * RPT + j * ZR, ZR), :])
        plsc.subcore_barrier()

        def chunk(j, carry):
            for b in range(NB):
                q = j * NB + b
                pltpu.make_async_copy(m_c.at[srcv.at[q]], bufs[b], sems[b]).wait()
                pltpu.sync_copy(bufs[b], acc.at[dstv.at[q]], add=True)

                @pl.when(q + NB < NCH2)
                def _():
                    pltpu.async_copy(m_c.at[srcv.at[q + NB]], bufs[b], sems[b])

            return carry

        lax.fori_loop(0, NCH2 // NB, chunk, 0)
        plsc.subcore_barrier()

        pltpu.sync_copy(
            acc.at[pl.ds(sid * RPT, RPT), :],
            out_hbm.at[cid, pl.ds(sid * RPT, RPT), :],
        )

    return agg


def _make_agg(F):
    @functools.partial(
        pl.kernel,
        mesh=_MESH,
        compiler_params=_SC_PARAMS,
        out_type=jax.ShapeDtypeStruct((NC, NP, F), jnp.float32),
        scratch_types=[
            pltpu.VMEM((NCH, C), jnp.int32),
            pltpu.VMEM((NCH, C), jnp.int32),
            pltpu.VMEM((C, F), jnp.float32),
            pltpu.VMEM((C, F), jnp.float32),
            pltpu.VMEM((C, F), jnp.float32),
            pltpu.VMEM((C, F), jnp.float32),
            pltpu.VMEM((ZR, F), jnp.float32),
            pltpu.VMEM_SHARED((NP, F), jnp.float32),
            pltpu.SemaphoreType.DMA,
            pltpu.SemaphoreType.DMA,
            pltpu.SemaphoreType.DMA,
            pltpu.SemaphoreType.DMA,
        ],
    )
    def agg(m_hbm, src_hbm, dst_hbm, out_hbm, srcv, dstv,
            r0, r1, r2, r3, zbuf, acc, s0, s1, s2, s3):
        cid = lax.axis_index("c")
        sid = lax.axis_index("s")
        wid = cid * NS + sid
        bufs = (r0, r1, r2, r3)
        sems = (s0, s1, s2, s3)
        NB = 4

        pltpu.sync_copy(src_hbm.at[wid], srcv)
        pltpu.sync_copy(dst_hbm.at[wid], dstv)

        for b in range(NB):
            pltpu.async_copy(m_hbm.at[srcv.at[b]], bufs[b], sems[b])

        zero16 = jnp.zeros((16,), jnp.float32)
        vpr = F // 16

        def zb(i, carry):
            zbuf[i // vpr, pl.ds((i % vpr) * 16, 16)] = zero16
            return carry

        lax.fori_loop(0, ZR * vpr, zb, 0)

        for j in range(RPT // ZR):
            pltpu.sync_copy(zbuf, acc.at[pl.ds(sid * RPT + j * ZR, ZR), :])
        plsc.subcore_barrier()

        def chunk(j, carry):
            for b in range(NB):
                q = j * NB + b
                pltpu.make_async_copy(m_hbm.at[srcv.at[q]], bufs[b], sems[b]).wait()
                pltpu.sync_copy(bufs[b], acc.at[dstv.at[q]], add=True)

                @pl.when(q + NB < NCH)
                def _():
                    pltpu.async_copy(m_hbm.at[srcv.at[q + NB]], bufs[b], sems[b])

            return carry

        lax.fori_loop(0, NCH // NB, chunk, 0)
        plsc.subcore_barrier()

        pltpu.sync_copy(
            acc.at[pl.ds(sid * RPT, RPT), :],
            out_hbm.at[cid, pl.ds(sid * RPT, RPT), :],
        )

    return agg


_agg128 = _make_agg_fsplit(64, jnp.bfloat16)
_agg64 = _make_agg(64)
_agg16 = _make_agg(16)


_B = 2000


def _tc0_body(degp_ref, x_ref, m1_ref, no_ref, ni_ref):
    deg = jnp.sum(degp_ref[...], axis=0)
    norm = jnp.where(deg > 0, lax.rsqrt(deg), 0.0)
    nT = jnp.transpose(norm)
    no = jnp.broadcast_to(nT[:, 0:1], (N, 128))
    ni = jnp.broadcast_to(nT[:, 1:2], (N, 128))
    no_ref[...] = no
    ni_ref[...] = ni
    m1 = (x_ref[...] * no).astype(jnp.bfloat16)
    m1_ref[0] = m1[:, :64]
    m1_ref[1] = m1[:, 64:]


def _tc0(degp, x):
    return pl.pallas_call(
        _tc0_body,
        grid=(1,),
        in_specs=[
            pl.BlockSpec((NW, 2, N), lambda i: (0, 0, 0)),
            pl.BlockSpec((N, 128), lambda i: (0, 0)),
        ],
        out_specs=[
            pl.BlockSpec((2, N, 64), lambda i: (0, 0, 0)),
            pl.BlockSpec((N, 128), lambda i: (0, 0)),
            pl.BlockSpec((N, 128), lambda i: (0, 0)),
        ],
        out_shape=[
            jax.ShapeDtypeStruct((2, N, 64), jnp.bfloat16),
            jax.ShapeDtypeStruct((N, 128), jnp.float32),
            jax.ShapeDtypeStruct((N, 128), jnp.float32),
        ],
    )(degp, x)


def _tc1_body(p_ref, ni_ref, no_ref, W1_ref, b1_ref, W2_ref, m2_ref):
    agg = jnp.concatenate([p_ref[0], p_ref[1]], axis=1).astype(jnp.float32) * ni_ref[...]
    h1 = jnp.maximum(
        jnp.dot(agg, W1_ref[...], preferred_element_type=jnp.float32)
        + b1_ref[...],
        0.0,
    )
    m2_ref[...] = (
        jnp.dot(h1, W2_ref[...], preferred_element_type=jnp.float32)
        * no_ref[...][:, :64]
    )


def _tc1(p1, ni, no, W1, b1, W2):
    return pl.pallas_call(
        _tc1_body,
        grid=(N // _B,),
        in_specs=[
            pl.BlockSpec((2, _B, 64), lambda i: (0, i, 0)),
            pl.BlockSpec((_B, 128), lambda i: (i, 0)),
            pl.BlockSpec((_B, 128), lambda i: (i, 0)),
            pl.BlockSpec((128, 128), lambda i: (0, 0)),
            pl.BlockSpec((1, 128), lambda i: (0, 0)),
            pl.BlockSpec((128, 64), lambda i: (0, 0)),
        ],
        out_specs=pl.BlockSpec((_B, 64), lambda i: (i, 0)),
        out_shape=jax.ShapeDtypeStruct((N, 64), jnp.float32),
    )(p1, ni, no, W1, b1, W2)


def _tc2_body(p_ref, ni_ref, no_ref, b2_ref, W3_ref, m3_ref):
    h2 = jnp.maximum(
        (p_ref[0] + p_ref[1]) * ni_ref[...][:, :64] + b2_ref[...], 0.0
    )
    m3_ref[...] = (
        jnp.dot(h2, W3_ref[...], preferred_element_type=jnp.float32)
        * no_ref[...][:, :16]
    )


def _tc2(p2, ni, no, b2, W3):
    return pl.pallas_call(
        _tc2_body,
        grid=(N // _B,),
        in_specs=[
            pl.BlockSpec((2, _B, 64), lambda i: (0, i, 0)),
            pl.BlockSpec((_B, 128), lambda i: (i, 0)),
            pl.BlockSpec((_B, 128), lambda i: (i, 0)),
            pl.BlockSpec((1, 64), lambda i: (0, 0)),
            pl.BlockSpec((64, 16), lambda i: (0, 0)),
        ],
        out_specs=pl.BlockSpec((_B, 16), lambda i: (i, 0)),
        out_shape=jax.ShapeDtypeStruct((N, 16), jnp.float32),
    )(p2, ni, no, b2, W3)


def _tc3_body(p_ref, ni_ref, b3_ref, h3_ref):
    h3_ref[...] = (p_ref[0] + p_ref[1]) * ni_ref[...][:, :16] + b3_ref[...]


def _tc3(p3, ni, b3):
    return pl.pallas_call(
        _tc3_body,
        grid=(N // _B,),
        in_specs=[
            pl.BlockSpec((2, _B, 16), lambda i: (0, i, 0)),
            pl.BlockSpec((_B, 128), lambda i: (i, 0)),
            pl.BlockSpec((1, 16), lambda i: (0, 0)),
        ],
        out_specs=pl.BlockSpec((_B, 16), lambda i: (i, 0)),
        out_shape=jax.ShapeDtypeStruct((N, 16), jnp.float32),
    )(p3, ni, b3)


def kernel(x, edge_index, W1, b1, W2, b2, W3, b3):
    src = edge_index[0].astype(jnp.int32)
    dst = edge_index[1].astype(jnp.int32)
    src_w = src.reshape(NW, NCH, C)
    dst_w = dst.reshape(NW, NCH, C)
    src_t = src.reshape(NS, NCH2, C)
    dst_t = dst.reshape(NS, NCH2, C)

    degp = _deg_kernel(src, dst)
    m1, no, ni = _tc0(degp.reshape(NW, 2, N), x)

    p1 = _agg128(m1, src_t, dst_t)
    m2 = _tc1(p1, ni, no, W1, b1.reshape(1, 128), W2)

    p2 = _agg64(m2, src_w, dst_w)
    m3 = _tc2(p2, ni, no, b2.reshape(1, 64), W3)

    p3 = _agg16(m3, src_w, dst_w)
    return _tc3(p3, ni, b3.reshape(1, 16))


# speedup vs baseline: 1.2037x; 1.0580x over previous
import functools

import jax
import jax.numpy as jnp
from jax import lax
from jax.experimental import pallas as pl
from jax.experimental.pallas import tpu as pltpu
from jax.experimental.pallas import tpu_sc as plsc

N = 10000
E = 320000
NC = 2
NS = 16
NW = NC * NS
EPW = E // NW
C = 125
NCH = EPW // C
NP = 10240
RPT = NP // NS
ZR = 128
EPT = E // NS
NCH2 = EPT // C

_MESH = plsc.VectorSubcoreMesh(core_axis_name="c", subcore_axis_name="s")
_SC_PARAMS = pltpu.CompilerParams(
    needs_layout_passes=False,
    use_tc_tiling_on_sc=False,
    skip_device_barrier=True,
)


@functools.partial(
    pl.kernel,
    mesh=_MESH,
    compiler_params=_SC_PARAMS,
    out_type=jax.ShapeDtypeStruct((NW * 2 * N,), jnp.float32),
    scratch_types=[
        pltpu.VMEM((EPW,), jnp.int32),
        pltpu.VMEM((N,), jnp.float32),
        pltpu.VMEM((N,), jnp.float32),
    ],
)
def _deg_kernel(src_hbm, dst_hbm, out_hbm, idx_v, dego_v, degi_v):
    cid = lax.axis_index("c")
    sid = lax.axis_index("s")
    wid = cid * NS + sid

    zero16 = jnp.zeros((16,), jnp.float32)

    def zloop(i, carry):
        dego_v[pl.ds(i * 16, 16)] = zero16
        degi_v[pl.ds(i * 16, 16)] = zero16
        return carry

    lax.fori_loop(0, N // 16, zloop, 0)

    ones16 = jnp.ones((16,), jnp.float32)

    pltpu.sync_copy(src_hbm.at[pl.ds(wid * EPW, EPW)], idx_v)

    def sloop(i, carry):
        s = idx_v[pl.ds(i * 16, 16)]
        plsc.addupdate_scatter(dego_v, [s], ones16)
        return carry

    lax.fori_loop(0, EPW // 16, sloop, 0)

    pltpu.sync_copy(dst_hbm.at[pl.ds(wid * EPW, EPW)], idx_v)

    def dloop(i, carry):
        t = idx_v[pl.ds(i * 16, 16)]
        plsc.addupdate_scatter(degi_v, [t], ones16)
        return carry

    lax.fori_loop(0, EPW // 16, dloop, 0)

    pltpu.sync_copy(dego_v, out_hbm.at[pl.ds(wid * 2 * N, N)])
    pltpu.sync_copy(degi_v, out_hbm.at[pl.ds(wid * 2 * N + N, N)])


def _make_agg_fsplit(H, dt=jnp.float32):
    @functools.partial(
        pl.kernel,
        mesh=_MESH,
        compiler_params=_SC_PARAMS,
        out_type=jax.ShapeDtypeStruct((NC, NP, H), dt),
        scratch_types=[
            pltpu.VMEM((NCH2, C), jnp.int32),
            pltpu.VMEM((NCH2, C), jnp.int32),
            pltpu.VMEM((C, H), dt),
            pltpu.VMEM((C, H), dt),
            pltpu.VMEM((C, H), dt),
            pltpu.VMEM((C, H), dt),
            pltpu.VMEM((ZR, H), dt),
            pltpu.VMEM_SHARED((NP, H), dt),
            pltpu.SemaphoreType.DMA,
            pltpu.SemaphoreType.DMA,
            pltpu.SemaphoreType.DMA,
            pltpu.SemaphoreType.DMA,
        ],
    )
    def agg(m_hbm, src_hbm, dst_hbm, out_hbm, srcv, dstv,
            r0, r1, r2, r3, zbuf, acc, s0, s1, s2, s3):
        cid = lax.axis_index("c")
        sid = lax.axis_index("s")
        bufs = (r0, r1, r2, r3)
        sems = (s0, s1, s2, s3)
        NB = 4
        lanes = 16 if dt == jnp.float32 else 32
        vpr = H // lanes

        pltpu.sync_copy(src_hbm.at[sid], srcv)
        pltpu.sync_copy(dst_hbm.at[sid], dstv)
        m_c = m_hbm.at[cid]

        for b in range(NB):
            pltpu.async_copy(m_c.at[srcv.at[b]], bufs[b], sems[b])

        zerov = jnp.zeros((lanes,), dt)

        def zb(i, carry):
            zbuf[i // vpr, pl.ds((i % vpr) * lanes, lanes)] = zerov
            return carry

        lax.fori_loop(0, ZR * vpr, zb, 0)

        for j in range(RPT // ZR):
            pltpu.sync_copy(zbuf, acc.at[pl.ds(sid * RPT + j * ZR, ZR), :])
        plsc.subcore_barrier()

        def chunk(j, carry):
            for b in range(NB):
                q = j * NB + b
                pltpu.make_async_copy(m_c.at[srcv.at[q]], bufs[b], sems[b]).wait()
                pltpu.sync_copy(bufs[b], acc.at[dstv.at[q]], add=True)

                @pl.when(q + NB < NCH2)
                def _():
                    pltpu.async_copy(m_c.at[srcv.at[q + NB]], bufs[b], sems[b])

            return carry

        lax.fori_loop(0, NCH2 // NB, chunk, 0)
        plsc.subcore_barrier()

        pltpu.sync_copy(
            acc.at[pl.ds(sid * RPT, RPT), :],
            out_hbm.at[cid, pl.ds(sid * RPT, RPT), :],
        )

    return agg


def _make_agg(F, dt=jnp.float32):
    @functools.partial(
        pl.kernel,
        mesh=_MESH,
        compiler_params=_SC_PARAMS,
        out_type=jax.ShapeDtypeStruct((NC, NP, F), dt),
        scratch_types=[
            pltpu.VMEM((NCH, C), jnp.int32),
            pltpu.VMEM((NCH, C), jnp.int32),
            pltpu.VMEM((C, F), dt),
            pltpu.VMEM((C, F), dt),
            pltpu.VMEM((C, F), dt),
            pltpu.VMEM((C, F), dt),
            pltpu.VMEM((ZR, F), dt),
            pltpu.VMEM_SHARED((NP, F), dt),
            pltpu.SemaphoreType.DMA,
            pltpu.SemaphoreType.DMA,
            pltpu.SemaphoreType.DMA,
            pltpu.SemaphoreType.DMA,
        ],
    )
    def agg(m_hbm, src_hbm, dst_hbm, out_hbm, srcv, dstv,
            r0, r1, r2, r3, zbuf, acc, s0, s1, s2, s3):
        cid = lax.axis_index("c")
        sid = lax.axis_index("s")
        wid = cid * NS + sid
        bufs = (r0, r1, r2, r3)
        sems = (s0, s1, s2, s3)
        NB = 4

        pltpu.sync_copy(src_hbm.at[wid], srcv)
        pltpu.sync_copy(dst_hbm.at[wid], dstv)

        for b in range(NB):
            pltpu.async_copy(m_hbm.at[srcv.at[b]], bufs[b], sems[b])

        lanes = 16 if dt == jnp.float32 else 32
        vpr = F // lanes
        zerov = jnp.zeros((lanes,), dt)

        def zb(i, carry):
            zbuf[i // vpr, pl.ds((i % vpr) * lanes, lanes)] = zerov
            return carry

        lax.fori_loop(0, ZR * vpr, zb, 0)

        for j in range(RPT // ZR):
            pltpu.sync_copy(zbuf, acc.at[pl.ds(sid * RPT + j * ZR, ZR), :])
        plsc.subcore_barrier()

        def chunk(j, carry):
            for b in range(NB):
                q = j * NB + b
                pltpu.make_async_copy(m_hbm.at[srcv.at[q]], bufs[b], sems[b]).wait()
                pltpu.sync_copy(bufs[b], acc.at[dstv.at[q]], add=True)

                @pl.when(q + NB < NCH)
                def _():
                    pltpu.async_copy(m_hbm.at[srcv.at[q + NB]], bufs[b], sems[b])

            return carry

        lax.fori_loop(0, NCH // NB, chunk, 0)
        plsc.subcore_barrier()

        pltpu.sync_copy(
            acc.at[pl.ds(sid * RPT, RPT), :],
            out_hbm.at[cid, pl.ds(sid * RPT, RPT), :],
        )

    return agg


_agg128 = _make_agg_fsplit(64, jnp.bfloat16)
_agg64 = _make_agg(64, jnp.bfloat16)
_agg16 = _make_agg(16)


_B = 2000


def _tc0_body(degp_ref, x_ref, m1_ref, no_ref, ni_ref):
    deg = jnp.sum(degp_ref[...], axis=0)
    norm = jnp.where(deg > 0, lax.rsqrt(deg), 0.0)
    nT = jnp.transpose(norm)
    no = jnp.broadcast_to(nT[:, 0:1], (N, 128))
    ni = jnp.broadcast_to(nT[:, 1:2], (N, 128))
    no_ref[...] = no
    ni_ref[...] = ni
    m1 = (x_ref[...] * no).astype(jnp.bfloat16)
    m1_ref[0] = m1[:, :64]
    m1_ref[1] = m1[:, 64:]


def _tc0(degp, x):
    return pl.pallas_call(
        _tc0_body,
        grid=(1,),
        in_specs=[
            pl.BlockSpec((NW, 2, N), lambda i: (0, 0, 0)),
            pl.BlockSpec((N, 128), lambda i: (0, 0)),
        ],
        out_specs=[
            pl.BlockSpec((2, N, 64), lambda i: (0, 0, 0)),
            pl.BlockSpec((N, 128), lambda i: (0, 0)),
            pl.BlockSpec((N, 128), lambda i: (0, 0)),
        ],
        out_shape=[
            jax.ShapeDtypeStruct((2, N, 64), jnp.bfloat16),
            jax.ShapeDtypeStruct((N, 128), jnp.float32),
            jax.ShapeDtypeStruct((N, 128), jnp.float32),
        ],
    )(degp, x)


def _tc1_body(p_ref, ni_ref, no_ref, W1_ref, b1_ref, W2_ref, m2_ref):
    agg = jnp.concatenate([p_ref[0], p_ref[1]], axis=1).astype(jnp.float32) * ni_ref[...]
    h1 = jnp.maximum(
        jnp.dot(agg, W1_ref[...], preferred_element_type=jnp.float32)
        + b1_ref[...],
        0.0,
    )
    m2_ref[...] = (
        jnp.dot(h1, W2_ref[...], preferred_element_type=jnp.float32)
        * no_ref[...][:, :64]
    ).astype(jnp.bfloat16)


def _tc1(p1, ni, no, W1, b1, W2):
    return pl.pallas_call(
        _tc1_body,
        grid=(N // _B,),
        in_specs=[
            pl.BlockSpec((2, _B, 64), lambda i: (0, i, 0)),
            pl.BlockSpec((_B, 128), lambda i: (i, 0)),
            pl.BlockSpec((_B, 128), lambda i: (i, 0)),
            pl.BlockSpec((128, 128), lambda i: (0, 0)),
            pl.BlockSpec((1, 128), lambda i: (0, 0)),
            pl.BlockSpec((128, 64), lambda i: (0, 0)),
        ],
        out_specs=pl.BlockSpec((_B, 64), lambda i: (i, 0)),
        out_shape=jax.ShapeDtypeStruct((N, 64), jnp.bfloat16),
    )(p1, ni, no, W1, b1, W2)


def _tc2_body(p_ref, ni_ref, no_ref, b2_ref, W3_ref, m3_ref):
    h2 = jnp.maximum(
        (p_ref[0].astype(jnp.float32) + p_ref[1].astype(jnp.float32))
        * ni_ref[...][:, :64]
        + b2_ref[...],
        0.0,
    )
    m3_ref[...] = (
        jnp.dot(h2, W3_ref[...], preferred_element_type=jnp.float32)
        * no_ref[...][:, :16]
    )


def _tc2(p2, ni, no, b2, W3):
    return pl.pallas_call(
        _tc2_body,
        grid=(N // _B,),
        in_specs=[
            pl.BlockSpec((2, _B, 64), lambda i: (0, i, 0)),
            pl.BlockSpec((_B, 128), lambda i: (i, 0)),
            pl.BlockSpec((_B, 128), lambda i: (i, 0)),
            pl.BlockSpec((1, 64), lambda i: (0, 0)),
            pl.BlockSpec((64, 16), lambda i: (0, 0)),
        ],
        out_specs=pl.BlockSpec((_B, 16), lambda i: (i, 0)),
        out_shape=jax.ShapeDtypeStruct((N, 16), jnp.float32),
    )(p2, ni, no, b2, W3)


def _tc3_body(p_ref, ni_ref, b3_ref, h3_ref):
    h3_ref[...] = (p_ref[0] + p_ref[1]) * ni_ref[...][:, :16] + b3_ref[...]


def _tc3(p3, ni, b3):
    return pl.pallas_call(
        _tc3_body,
        grid=(N // _B,),
        in_specs=[
            pl.BlockSpec((2, _B, 16), lambda i: (0, i, 0)),
            pl.BlockSpec((_B, 128), lambda i: (i, 0)),
            pl.BlockSpec((1, 16), lambda i: (0, 0)),
        ],
        out_specs=pl.BlockSpec((_B, 16), lambda i: (i, 0)),
        out_shape=jax.ShapeDtypeStruct((N, 16), jnp.float32),
    )(p3, ni, b3)


def kernel(x, edge_index, W1, b1, W2, b2, W3, b3):
    src = edge_index[0].astype(jnp.int32)
    dst = edge_index[1].astype(jnp.int32)
    src_w = src.reshape(NW, NCH, C)
    dst_w = dst.reshape(NW, NCH, C)
    src_t = src.reshape(NS, NCH2, C)
    dst_t = dst.reshape(NS, NCH2, C)

    degp = _deg_kernel(src, dst)
    m1, no, ni = _tc0(degp.reshape(NW, 2, N), x)

    p1 = _agg128(m1, src_t, dst_t)
    m2 = _tc1(p1, ni, no, W1, b1.reshape(1, 128), W2)

    p2 = _agg64(m2, src_w, dst_w)
    m3 = _tc2(p2, ni, no, b2.reshape(1, 64), W3)

    p3 = _agg16(m3, src_w, dst_w)
    return _tc3(p3, ni, b3.reshape(1, 16))


# speedup vs baseline: 1.2463x; 1.0354x over previous
import functools

import jax
import jax.numpy as jnp
from jax import lax
from jax.experimental import pallas as pl
from jax.experimental.pallas import tpu as pltpu
from jax.experimental.pallas import tpu_sc as plsc

N = 10000
E = 320000
NC = 2
NS = 16
NW = NC * NS
EPW = E // NW
C = 125
NCH = EPW // C
NP = 10240
RPT = NP // NS
ZR = 128
EPT = E // NS
NCH2 = EPT // C

_MESH = plsc.VectorSubcoreMesh(core_axis_name="c", subcore_axis_name="s")
_SC_PARAMS = pltpu.CompilerParams(
    needs_layout_passes=False,
    use_tc_tiling_on_sc=False,
    skip_device_barrier=True,
)


@functools.partial(
    pl.kernel,
    mesh=_MESH,
    compiler_params=_SC_PARAMS,
    out_type=jax.ShapeDtypeStruct((NW * 2 * N,), jnp.float32),
    scratch_types=[
        pltpu.VMEM((EPW,), jnp.int32),
        pltpu.VMEM((N,), jnp.float32),
        pltpu.VMEM((N,), jnp.float32),
    ],
)
def _deg_kernel(src_hbm, dst_hbm, out_hbm, idx_v, dego_v, degi_v):
    cid = lax.axis_index("c")
    sid = lax.axis_index("s")
    wid = cid * NS + sid

    zero16 = jnp.zeros((16,), jnp.float32)

    def zloop(i, carry):
        dego_v[pl.ds(i * 16, 16)] = zero16
        degi_v[pl.ds(i * 16, 16)] = zero16
        return carry

    lax.fori_loop(0, N // 16, zloop, 0, unroll=4)

    ones16 = jnp.ones((16,), jnp.float32)

    pltpu.sync_copy(src_hbm.at[pl.ds(wid * EPW, EPW)], idx_v)

    def sloop(i, carry):
        s = idx_v[pl.ds(i * 16, 16)]
        plsc.addupdate_scatter(dego_v, [s], ones16)
        return carry

    lax.fori_loop(0, EPW // 16, sloop, 0, unroll=4)

    pltpu.sync_copy(dst_hbm.at[pl.ds(wid * EPW, EPW)], idx_v)

    def dloop(i, carry):
        t = idx_v[pl.ds(i * 16, 16)]
        plsc.addupdate_scatter(degi_v, [t], ones16)
        return carry

    lax.fori_loop(0, EPW // 16, dloop, 0, unroll=4)

    pltpu.sync_copy(dego_v, out_hbm.at[pl.ds(wid * 2 * N, N)])
    pltpu.sync_copy(degi_v, out_hbm.at[pl.ds(wid * 2 * N + N, N)])


def _make_agg_fsplit(H, dt=jnp.float32):
    @functools.partial(
        pl.kernel,
        mesh=_MESH,
        compiler_params=_SC_PARAMS,
        out_type=jax.ShapeDtypeStruct((NC, NP, H), dt),
        scratch_types=[
            pltpu.VMEM((NCH2, C), jnp.int32),
            pltpu.VMEM((NCH2, C), jnp.int32),
            pltpu.VMEM((C, H), dt),
            pltpu.VMEM((C, H), dt),
            pltpu.VMEM((C, H), dt),
            pltpu.VMEM((C, H), dt),
            pltpu.VMEM((ZR, H), dt),
            pltpu.VMEM_SHARED((NP, H), dt),
            pltpu.SemaphoreType.DMA,
            pltpu.SemaphoreType.DMA,
            pltpu.SemaphoreType.DMA,
            pltpu.SemaphoreType.DMA,
        ],
    )
    def agg(m_hbm, src_hbm, dst_hbm, out_hbm, srcv, dstv,
            r0, r1, r2, r3, zbuf, acc, s0, s1, s2, s3):
        cid = lax.axis_index("c")
        sid = lax.axis_index("s")
        bufs = (r0, r1, r2, r3)
        sems = (s0, s1, s2, s3)
        NB = 4
        lanes = 16 if dt == jnp.float32 else 32
        vpr = H // lanes

        pltpu.sync_copy(src_hbm.at[sid], srcv)
        pltpu.sync_copy(dst_hbm.at[sid], dstv)
        m_c = m_hbm.at[cid]

        for b in range(NB):
            pltpu.async_copy(m_c.at[srcv.at[b]], bufs[b], sems[b])

        zerov = jnp.zeros((lanes,), dt)

        def zb(i, carry):
            zbuf[i // vpr, pl.ds((i % vpr) * lanes, lanes)] = zerov
            return carry

        lax.fori_loop(0, ZR * vpr, zb, 0)

        for j in range(RPT // ZR):
            pltpu.sync_copy(zbuf, acc.at[pl.ds(sid * RPT + j * ZR, ZR), :])
        plsc.subcore_barrier()

        def chunk(j, carry):
            for b in range(NB):
                q = j * NB + b
                pltpu.make_async_copy(m_c.at[srcv.at[q]], bufs[b], sems[b]).wait()
                pltpu.sync_copy(bufs[b], acc.at[dstv.at[q]], add=True)

                @pl.when(q + NB < NCH2)
                def _():
                    pltpu.async_copy(m_c.at[srcv.at[q + NB]], bufs[b], sems[b])

            return carry

        lax.fori_loop(0, NCH2 // NB, chunk, 0)
        plsc.subcore_barrier()

        pltpu.sync_copy(
            acc.at[pl.ds(sid * RPT, RPT), :],
            out_hbm.at[cid, pl.ds(sid * RPT, RPT), :],
        )

    return agg


def _make_agg(F, dt=jnp.float32):
    @functools.partial(
        pl.kernel,
        mesh=_MESH,
        compiler_params=_SC_PARAMS,
        out_type=jax.ShapeDtypeStruct((NC, NP, F), dt),
        scratch_types=[
            pltpu.VMEM((NCH, C), jnp.int32),
            pltpu.VMEM((NCH, C), jnp.int32),
            pltpu.VMEM((C, F), dt),
            pltpu.VMEM((C, F), dt),
            pltpu.VMEM((C, F), dt),
            pltpu.VMEM((C, F), dt),
            pltpu.VMEM((ZR, F), dt),
            pltpu.VMEM_SHARED((NP, F), dt),
            pltpu.SemaphoreType.DMA,
            pltpu.SemaphoreType.DMA,
            pltpu.SemaphoreType.DMA,
            pltpu.SemaphoreType.DMA,
        ],
    )
    def agg(m_hbm, src_hbm, dst_hbm, out_hbm, srcv, dstv,
            r0, r1, r2, r3, zbuf, acc, s0, s1, s2, s3):
        cid = lax.axis_index("c")
        sid = lax.axis_index("s")
        bufs = (r0, r1, r2, r3)
        sems = (s0, s1, s2, s3)
        NB = 4

        pltpu.sync_copy(src_hbm.at[sid, pl.ds(cid * NCH, NCH)], srcv)
        pltpu.sync_copy(dst_hbm.at[sid, pl.ds(cid * NCH, NCH)], dstv)

        for b in range(NB):
            pltpu.async_copy(m_hbm.at[srcv.at[b]], bufs[b], sems[b])

        lanes = 16 if dt == jnp.float32 else 32
        vpr = F // lanes
        zerov = jnp.zeros((lanes,), dt)

        def zb(i, carry):
            zbuf[i // vpr, pl.ds((i % vpr) * lanes, lanes)] = zerov
            return carry

        lax.fori_loop(0, ZR * vpr, zb, 0)

        for j in range(RPT // ZR):
            pltpu.sync_copy(zbuf, acc.at[pl.ds(sid * RPT + j * ZR, ZR), :])
        plsc.subcore_barrier()

        def chunk(j, carry):
            for b in range(NB):
                q = j * NB + b
                pltpu.make_async_copy(m_hbm.at[srcv.at[q]], bufs[b], sems[b]).wait()
                pltpu.sync_copy(bufs[b], acc.at[dstv.at[q]], add=True)

                @pl.when(q + NB < NCH)
                def _():
                    pltpu.async_copy(m_hbm.at[srcv.at[q + NB]], bufs[b], sems[b])

            return carry

        lax.fori_loop(0, NCH // NB, chunk, 0)
        plsc.subcore_barrier()

        pltpu.sync_copy(
            acc.at[pl.ds(sid * RPT, RPT), :],
            out_hbm.at[cid, pl.ds(sid * RPT, RPT), :],
        )

    return agg


_agg128 = _make_agg_fsplit(64, jnp.bfloat16)
_agg64 = _make_agg(64, jnp.bfloat16)
_agg16 = _make_agg(16)


_B = 2000


def _tc0_body(degp_ref, x_ref, m1_ref, no_ref, ni_ref):
    dego = jnp.zeros((N,), jnp.float32)
    degi = jnp.zeros((N,), jnp.float32)
    for w in range(NW):
        dego = dego + degp_ref[pl.ds(w * 2 * N, N)]
        degi = degi + degp_ref[pl.ds(w * 2 * N + N, N)]
    deg = jnp.stack([dego, degi], axis=0)
    norm = jnp.where(deg > 0, lax.rsqrt(deg), 0.0)
    nT = jnp.transpose(norm)
    no = jnp.broadcast_to(nT[:, 0:1], (N, 128))
    ni = jnp.broadcast_to(nT[:, 1:2], (N, 128))
    no_ref[...] = no
    ni_ref[...] = ni
    m1 = (x_ref[...] * no).astype(jnp.bfloat16)
    m1_ref[0] = m1[:, :64]
    m1_ref[1] = m1[:, 64:]


def _tc0(degp, x):
    return pl.pallas_call(
        _tc0_body,
        grid=(1,),
        in_specs=[
            pl.BlockSpec((NW * 2 * N,), lambda i: (0,)),
            pl.BlockSpec((N, 128), lambda i: (0, 0)),
        ],
        out_specs=[
            pl.BlockSpec((2, N, 64), lambda i: (0, 0, 0)),
            pl.BlockSpec((N, 128), lambda i: (0, 0)),
            pl.BlockSpec((N, 128), lambda i: (0, 0)),
        ],
        out_shape=[
            jax.ShapeDtypeStruct((2, N, 64), jnp.bfloat16),
            jax.ShapeDtypeStruct((N, 128), jnp.float32),
            jax.ShapeDtypeStruct((N, 128), jnp.float32),
        ],
    )(degp, x)


def _tc1_body(p_ref, ni_ref, no_ref, W1_ref, b1_ref, W2_ref, m2_ref):
    agg = jnp.concatenate([p_ref[0], p_ref[1]], axis=1).astype(jnp.float32) * ni_ref[...]
    h1 = jnp.maximum(
        jnp.dot(agg, W1_ref[...], preferred_element_type=jnp.float32)
        + b1_ref[...],
        0.0,
    )
    m2_ref[...] = (
        jnp.dot(h1, W2_ref[...], preferred_element_type=jnp.float32)
        * no_ref[...][:, :64]
    ).astype(jnp.bfloat16)


def _tc1(p1, ni, no, W1, b1, W2):
    return pl.pallas_call(
        _tc1_body,
        grid=(N // _B,),
        in_specs=[
            pl.BlockSpec((2, _B, 64), lambda i: (0, i, 0)),
            pl.BlockSpec((_B, 128), lambda i: (i, 0)),
            pl.BlockSpec((_B, 128), lambda i: (i, 0)),
            pl.BlockSpec((128, 128), lambda i: (0, 0)),
            pl.BlockSpec((1, 128), lambda i: (0, 0)),
            pl.BlockSpec((128, 64), lambda i: (0, 0)),
        ],
        out_specs=pl.BlockSpec((_B, 64), lambda i: (i, 0)),
        out_shape=jax.ShapeDtypeStruct((N, 64), jnp.bfloat16),
    )(p1, ni, no, W1, b1, W2)


def _tc2_body(p_ref, ni_ref, no_ref, b2_ref, W3_ref, m3_ref):
    h2 = jnp.maximum(
        (p_ref[0].astype(jnp.float32) + p_ref[1].astype(jnp.float32))
        * ni_ref[...][:, :64]
        + b2_ref[...],
        0.0,
    )
    m3_ref[...] = (
        jnp.dot(h2, W3_ref[...], preferred_element_type=jnp.float32)
        * no_ref[...][:, :16]
    )


def _tc2(p2, ni, no, b2, W3):
    return pl.pallas_call(
        _tc2_body,
        grid=(N // _B,),
        in_specs=[
            pl.BlockSpec((2, _B, 64), lambda i: (0, i, 0)),
            pl.BlockSpec((_B, 128), lambda i: (i, 0)),
            pl.BlockSpec((_B, 128), lambda i: (i, 0)),
            pl.BlockSpec((1, 64), lambda i: (0, 0)),
            pl.BlockSpec((64, 16), lambda i: (0, 0)),
        ],
        out_specs=pl.BlockSpec((_B, 16), lambda i: (i, 0)),
        out_shape=jax.ShapeDtypeStruct((N, 16), jnp.float32),
    )(p2, ni, no, b2, W3)


def _tc3_body(p_ref, ni_ref, b3_ref, h3_ref):
    h3_ref[...] = (p_ref[0] + p_ref[1]) * ni_ref[...][:, :16] + b3_ref[...]


def _tc3(p3, ni, b3):
    return pl.pallas_call(
        _tc3_body,
        grid=(N // _B,),
        in_specs=[
            pl.BlockSpec((2, _B, 16), lambda i: (0, i, 0)),
            pl.BlockSpec((_B, 128), lambda i: (i, 0)),
            pl.BlockSpec((1, 16), lambda i: (0, 0)),
        ],
        out_specs=pl.BlockSpec((_B, 16), lambda i: (i, 0)),
        out_shape=jax.ShapeDtypeStruct((N, 16), jnp.float32),
    )(p3, ni, b3)


def kernel(x, edge_index, W1, b1, W2, b2, W3, b3):
    src = edge_index[0].astype(jnp.int32)
    dst = edge_index[1].astype(jnp.int32)
    src_t = src.reshape(NS, NCH2, C)
    dst_t = dst.reshape(NS, NCH2, C)

    degp = _deg_kernel(src, dst)
    m1, no, ni = _tc0(degp, x)

    p1 = _agg128(m1, src_t, dst_t)
    m2 = _tc1(p1, ni, no, W1, b1.reshape(1, 128), W2)

    p2 = _agg64(m2, src_t, dst_t)
    m3 = _tc2(p2, ni, no, b2.reshape(1, 64), W3)

    p3 = _agg16(m3, src_t, dst_t)
    return _tc3(p3, ni, b3.reshape(1, 16))


# speedup vs baseline: 1.2831x; 1.0295x over previous
import functools

import jax
import jax.numpy as jnp
from jax import lax
from jax.experimental import pallas as pl
from jax.experimental.pallas import tpu as pltpu
from jax.experimental.pallas import tpu_sc as plsc

N = 10000
E = 320000
NC = 2
NS = 16
NW = NC * NS
EPW = E // NW
C = 125
NCH = EPW // C
NP = 10240
RPT = NP // NS
ZR = 128
EPT = E // NS
NCH2 = EPT // C

_MESH = plsc.VectorSubcoreMesh(core_axis_name="c", subcore_axis_name="s")
_SC_PARAMS = pltpu.CompilerParams(
    needs_layout_passes=False,
    use_tc_tiling_on_sc=False,
    skip_device_barrier=True,
)


@functools.partial(
    pl.kernel,
    mesh=_MESH,
    compiler_params=_SC_PARAMS,
    out_type=jax.ShapeDtypeStruct((NW * 2 * N,), jnp.float32),
    scratch_types=[
        pltpu.VMEM((EPW,), jnp.int32),
        pltpu.VMEM((N,), jnp.float32),
        pltpu.VMEM((N,), jnp.float32),
    ],
)
def _deg_kernel(src_hbm, dst_hbm, out_hbm, idx_v, dego_v, degi_v):
    cid = lax.axis_index("c")
    sid = lax.axis_index("s")
    wid = cid * NS + sid

    zero16 = jnp.zeros((16,), jnp.float32)

    def zloop(i, carry):
        dego_v[pl.ds(i * 16, 16)] = zero16
        degi_v[pl.ds(i * 16, 16)] = zero16
        return carry

    lax.fori_loop(0, N // 16, zloop, 0, unroll=4)

    ones16 = jnp.ones((16,), jnp.float32)

    pltpu.sync_copy(src_hbm.at[pl.ds(wid * EPW, EPW)], idx_v)

    def sloop(i, carry):
        s = idx_v[pl.ds(i * 16, 16)]
        plsc.addupdate_scatter(dego_v, [s], ones16)
        return carry

    lax.fori_loop(0, EPW // 16, sloop, 0, unroll=4)

    pltpu.sync_copy(dst_hbm.at[pl.ds(wid * EPW, EPW)], idx_v)

    def dloop(i, carry):
        t = idx_v[pl.ds(i * 16, 16)]
        plsc.addupdate_scatter(degi_v, [t], ones16)
        return carry

    lax.fori_loop(0, EPW // 16, dloop, 0, unroll=4)

    pltpu.sync_copy(dego_v, out_hbm.at[pl.ds(wid * 2 * N, N)])
    pltpu.sync_copy(degi_v, out_hbm.at[pl.ds(wid * 2 * N + N, N)])


def _make_agg_fsplit(H, dt=jnp.float32):
    @functools.partial(
        pl.kernel,
        mesh=_MESH,
        compiler_params=_SC_PARAMS,
        out_type=jax.ShapeDtypeStruct((NC, NP, H), dt),
        scratch_types=[
            pltpu.VMEM((NCH2, C), jnp.int32),
            pltpu.VMEM((NCH2, C), jnp.int32),
            pltpu.VMEM((C, H), dt),
            pltpu.VMEM((C, H), dt),
            pltpu.VMEM((C, H), dt),
            pltpu.VMEM((C, H), dt),
            pltpu.VMEM((ZR, H), dt),
            pltpu.VMEM_SHARED((NP, H), dt),
            pltpu.SemaphoreType.DMA,
            pltpu.SemaphoreType.DMA,
            pltpu.SemaphoreType.DMA,
            pltpu.SemaphoreType.DMA,
        ],
    )
    def agg(m_hbm, src_hbm, dst_hbm, out_hbm, srcv, dstv,
            r0, r1, r2, r3, zbuf, acc, s0, s1, s2, s3):
        cid = lax.axis_index("c")
        sid = lax.axis_index("s")
        bufs = (r0, r1, r2, r3)
        sems = (s0, s1, s2, s3)
        NB = 4
        lanes = 16 if dt == jnp.float32 else 32
        vpr = H // lanes

        pltpu.sync_copy(src_hbm.at[sid], srcv)
        pltpu.sync_copy(dst_hbm.at[sid], dstv)
        m_c = m_hbm.at[cid]

        for b in range(NB):
            pltpu.async_copy(m_c.at[srcv.at[b]], bufs[b], sems[b])

        zerov = jnp.zeros((lanes,), dt)

        def zb(i, carry):
            zbuf[i // vpr, pl.ds((i % vpr) * lanes, lanes)] = zerov
            return carry

        lax.fori_loop(0, ZR * vpr, zb, 0)

        for j in range(RPT // ZR):
            pltpu.sync_copy(zbuf, acc.at[pl.ds(sid * RPT + j * ZR, ZR), :])
        plsc.subcore_barrier()

        def chunk(j, carry):
            for b in range(NB):
                q = j * NB + b
                pltpu.make_async_copy(m_c.at[srcv.at[q]], bufs[b], sems[b]).wait()
                pltpu.sync_copy(bufs[b], acc.at[dstv.at[q]], add=True)

                @pl.when(q + NB < NCH2)
                def _():
                    pltpu.async_copy(m_c.at[srcv.at[q + NB]], bufs[b], sems[b])

            return carry

        lax.fori_loop(0, NCH2 // NB, chunk, 0)
        plsc.subcore_barrier()

        pltpu.sync_copy(
            acc.at[pl.ds(sid * RPT, RPT), :],
            out_hbm.at[cid, pl.ds(sid * RPT, RPT), :],
        )

    return agg


def _make_agg(F, dt=jnp.float32, nb=4):
    @functools.partial(
        pl.kernel,
        mesh=_MESH,
        compiler_params=_SC_PARAMS,
        out_type=jax.ShapeDtypeStruct((NC, NP, F), dt),
        scratch_types=(
            [
                pltpu.VMEM((NCH, C), jnp.int32),
                pltpu.VMEM((NCH, C), jnp.int32),
            ]
            + [pltpu.VMEM((C, F), dt) for _ in range(nb)]
            + [pltpu.VMEM((ZR, F), dt)]
            + [pltpu.VMEM_SHARED((NP, F), dt)]
            + [pltpu.SemaphoreType.DMA for _ in range(nb)]
        ),
    )
    def agg(m_hbm, src_hbm, dst_hbm, out_hbm, srcv, dstv, *scr):
        bufs = scr[:nb]
        zbuf = scr[nb]
        acc = scr[nb + 1]
        sems = scr[nb + 2:]
        cid = lax.axis_index("c")
        sid = lax.axis_index("s")
        NB = nb

        pltpu.sync_copy(src_hbm.at[sid, pl.ds(cid * NCH, NCH)], srcv)
        pltpu.sync_copy(dst_hbm.at[sid, pl.ds(cid * NCH, NCH)], dstv)

        for b in range(NB):
            pltpu.async_copy(m_hbm.at[srcv.at[b]], bufs[b], sems[b])

        lanes = 16 if dt == jnp.float32 else 32
        vpr = F // lanes
        zerov = jnp.zeros((lanes,), dt)

        def zb(i, carry):
            zbuf[i // vpr, pl.ds((i % vpr) * lanes, lanes)] = zerov
            return carry

        lax.fori_loop(0, ZR * vpr, zb, 0)

        for j in range(RPT // ZR):
            pltpu.sync_copy(zbuf, acc.at[pl.ds(sid * RPT + j * ZR, ZR), :])
        plsc.subcore_barrier()

        def chunk(j, carry):
            for b in range(NB):
                q = j * NB + b
                pltpu.make_async_copy(m_hbm.at[srcv.at[q]], bufs[b], sems[b]).wait()
                pltpu.sync_copy(bufs[b], acc.at[dstv.at[q]], add=True)

                @pl.when(q + NB < NCH)
                def _():
                    pltpu.async_copy(m_hbm.at[srcv.at[q + NB]], bufs[b], sems[b])

            return carry

        lax.fori_loop(0, NCH // NB, chunk, 0)
        plsc.subcore_barrier()

        pltpu.sync_copy(
            acc.at[pl.ds(sid * RPT, RPT), :],
            out_hbm.at[cid, pl.ds(sid * RPT, RPT), :],
        )

    return agg


_agg128 = _make_agg_fsplit(64, jnp.bfloat16)
_agg64 = _make_agg(64, jnp.bfloat16)
_agg16 = _make_agg(16, nb=8)


_B = 2000


def _tc0_body(degp_ref, x_ref, m1_ref, no_ref, ni_ref):
    dego = jnp.zeros((N,), jnp.float32)
    degi = jnp.zeros((N,), jnp.float32)
    for w in range(NW):
        dego = dego + degp_ref[pl.ds(w * 2 * N, N)]
        degi = degi + degp_ref[pl.ds(w * 2 * N + N, N)]
    deg = jnp.stack([dego, degi], axis=0)
    norm = jnp.where(deg > 0, lax.rsqrt(deg), 0.0)
    nT = jnp.transpose(norm)
    no = jnp.broadcast_to(nT[:, 0:1], (N, 128))
    ni = jnp.broadcast_to(nT[:, 1:2], (N, 128))
    no_ref[...] = no
    ni_ref[...] = ni
    m1 = (x_ref[...] * no).astype(jnp.bfloat16)
    m1_ref[0] = m1[:, :64]
    m1_ref[1] = m1[:, 64:]


def _tc0(degp, x):
    return pl.pallas_call(
        _tc0_body,
        grid=(1,),
        in_specs=[
            pl.BlockSpec((NW * 2 * N,), lambda i: (0,)),
            pl.BlockSpec((N, 128), lambda i: (0, 0)),
        ],
        out_specs=[
            pl.BlockSpec((2, N, 64), lambda i: (0, 0, 0)),
            pl.BlockSpec((N, 128), lambda i: (0, 0)),
            pl.BlockSpec((N, 128), lambda i: (0, 0)),
        ],
        out_shape=[
            jax.ShapeDtypeStruct((2, N, 64), jnp.bfloat16),
            jax.ShapeDtypeStruct((N, 128), jnp.float32),
            jax.ShapeDtypeStruct((N, 128), jnp.float32),
        ],
    )(degp, x)


def _tc1_body(p_ref, ni_ref, no_ref, W1_ref, b1_ref, W2_ref, m2_ref):
    agg = jnp.concatenate([p_ref[0], p_ref[1]], axis=1).astype(jnp.float32) * ni_ref[...]
    h1 = jnp.maximum(
        jnp.dot(agg, W1_ref[...], preferred_element_type=jnp.float32)
        + b1_ref[...],
        0.0,
    )
    m2_ref[...] = (
        jnp.dot(h1, W2_ref[...], preferred_element_type=jnp.float32)
        * no_ref[...][:, :64]
    ).astype(jnp.bfloat16)


def _tc1(p1, ni, no, W1, b1, W2):
    return pl.pallas_call(
        _tc1_body,
        grid=(N // _B,),
        in_specs=[
            pl.BlockSpec((2, _B, 64), lambda i: (0, i, 0)),
            pl.BlockSpec((_B, 128), lambda i: (i, 0)),
            pl.BlockSpec((_B, 128), lambda i: (i, 0)),
            pl.BlockSpec((128, 128), lambda i: (0, 0)),
            pl.BlockSpec((1, 128), lambda i: (0, 0)),
            pl.BlockSpec((128, 64), lambda i: (0, 0)),
        ],
        out_specs=pl.BlockSpec((_B, 64), lambda i: (i, 0)),
        out_shape=jax.ShapeDtypeStruct((N, 64), jnp.bfloat16),
    )(p1, ni, no, W1, b1, W2)


def _tc2_body(p_ref, ni_ref, no_ref, b2_ref, W3_ref, m3_ref):
    h2 = jnp.maximum(
        (p_ref[0].astype(jnp.float32) + p_ref[1].astype(jnp.float32))
        * ni_ref[...][:, :64]
        + b2_ref[...],
        0.0,
    )
    m3_ref[...] = (
        jnp.dot(h2, W3_ref[...], preferred_element_type=jnp.float32)
        * no_ref[...][:, :16]
    )


def _tc2(p2, ni, no, b2, W3):
    return pl.pallas_call(
        _tc2_body,
        grid=(N // _B,),
        in_specs=[
            pl.BlockSpec((2, _B, 64), lambda i: (0, i, 0)),
            pl.BlockSpec((_B, 128), lambda i: (i, 0)),
            pl.BlockSpec((_B, 128), lambda i: (i, 0)),
            pl.BlockSpec((1, 64), lambda i: (0, 0)),
            pl.BlockSpec((64, 16), lambda i: (0, 0)),
        ],
        out_specs=pl.BlockSpec((_B, 16), lambda i: (i, 0)),
        out_shape=jax.ShapeDtypeStruct((N, 16), jnp.float32),
    )(p2, ni, no, b2, W3)


def _tc3_body(p_ref, ni_ref, b3_ref, h3_ref):
    h3_ref[...] = (p_ref[0] + p_ref[1]) * ni_ref[...][:, :16] + b3_ref[...]


def _tc3(p3, ni, b3):
    return pl.pallas_call(
        _tc3_body,
        grid=(N // _B,),
        in_specs=[
            pl.BlockSpec((2, _B, 16), lambda i: (0, i, 0)),
            pl.BlockSpec((_B, 128), lambda i: (i, 0)),
            pl.BlockSpec((1, 16), lambda i: (0, 0)),
        ],
        out_specs=pl.BlockSpec((_B, 16), lambda i: (i, 0)),
        out_shape=jax.ShapeDtypeStruct((N, 16), jnp.float32),
    )(p3, ni, b3)


def kernel(x, edge_index, W1, b1, W2, b2, W3, b3):
    src = edge_index[0].astype(jnp.int32)
    dst = edge_index[1].astype(jnp.int32)
    src_t = src.reshape(NS, NCH2, C)
    dst_t = dst.reshape(NS, NCH2, C)

    degp = _deg_kernel(src, dst)
    m1, no, ni = _tc0(degp, x)

    p1 = _agg128(m1, src_t, dst_t)
    m2 = _tc1(p1, ni, no, W1, b1.reshape(1, 128), W2)

    p2 = _agg64(m2, src_t, dst_t)
    m3 = _tc2(p2, ni, no, b2.reshape(1, 64), W3)

    p3 = _agg16(m3, src_t, dst_t)
    return _tc3(p3, ni, b3.reshape(1, 16))


# speedup vs baseline: 1.3316x; 1.0378x over previous
import functools

import jax
import jax.numpy as jnp
from jax import lax
from jax.experimental import pallas as pl
from jax.experimental.pallas import tpu as pltpu
from jax.experimental.pallas import tpu_sc as plsc

N = 10000
E = 320000
NC = 2
NS = 16
NW = NC * NS
EPW = E // NW
C = 125
NCH = EPW // C
NP = 10240
RPT = NP // NS
ZR = 128
EPT = E // NS
NCH2 = EPT // C

_MESH = plsc.VectorSubcoreMesh(core_axis_name="c", subcore_axis_name="s")
_SC_PARAMS = pltpu.CompilerParams(
    needs_layout_passes=False,
    use_tc_tiling_on_sc=False,
    skip_device_barrier=True,
)


@functools.partial(
    pl.kernel,
    mesh=_MESH,
    compiler_params=_SC_PARAMS,
    out_type=jax.ShapeDtypeStruct((NW * 2 * N,), jnp.float32),
    scratch_types=[
        pltpu.VMEM((EPW,), jnp.int32),
        pltpu.VMEM((N,), jnp.float32),
        pltpu.VMEM((N,), jnp.float32),
    ],
)
def _deg_kernel(src_hbm, dst_hbm, out_hbm, idx_v, dego_v, degi_v):
    cid = lax.axis_index("c")
    sid = lax.axis_index("s")
    wid = cid * NS + sid

    zero16 = jnp.zeros((16,), jnp.float32)

    def zloop(i, carry):
        dego_v[pl.ds(i * 16, 16)] = zero16
        degi_v[pl.ds(i * 16, 16)] = zero16
        return carry

    lax.fori_loop(0, N // 16, zloop, 0, unroll=4)

    ones16 = jnp.ones((16,), jnp.float32)

    pltpu.sync_copy(src_hbm.at[pl.ds(wid * EPW, EPW)], idx_v)

    def sloop(i, carry):
        s = idx_v[pl.ds(i * 16, 16)]
        plsc.addupdate_scatter(dego_v, [s], ones16)
        return carry

    lax.fori_loop(0, EPW // 16, sloop, 0, unroll=4)

    pltpu.sync_copy(dst_hbm.at[pl.ds(wid * EPW, EPW)], idx_v)

    def dloop(i, carry):
        t = idx_v[pl.ds(i * 16, 16)]
        plsc.addupdate_scatter(degi_v, [t], ones16)
        return carry

    lax.fori_loop(0, EPW // 16, dloop, 0, unroll=4)

    pltpu.sync_copy(dego_v, out_hbm.at[pl.ds(wid * 2 * N, N)])
    pltpu.sync_copy(degi_v, out_hbm.at[pl.ds(wid * 2 * N + N, N)])


def _make_agg_fsplit(H, dt=jnp.float32, nb=4):
    @functools.partial(
        pl.kernel,
        mesh=_MESH,
        compiler_params=_SC_PARAMS,
        out_type=jax.ShapeDtypeStruct((NC, NP, H), dt),
        scratch_types=(
            [
                pltpu.VMEM((NCH2, C), jnp.int32),
                pltpu.VMEM((NCH2, C), jnp.int32),
            ]
            + [pltpu.VMEM((C, H), dt) for _ in range(nb)]
            + [pltpu.VMEM((ZR, H), dt)]
            + [pltpu.VMEM_SHARED((NP, H), dt)]
            + [pltpu.SemaphoreType.DMA for _ in range(nb)]
        ),
    )
    def agg(m_hbm, src_hbm, dst_hbm, out_hbm, srcv, dstv, *scr):
        bufs = scr[:nb]
        zbuf = scr[nb]
        acc = scr[nb + 1]
        sems = scr[nb + 2:]
        cid = lax.axis_index("c")
        sid = lax.axis_index("s")
        NB = nb
        lanes = 16 if dt == jnp.float32 else 32
        vpr = H // lanes

        pltpu.sync_copy(src_hbm.at[sid], srcv)
        pltpu.sync_copy(dst_hbm.at[sid], dstv)
        m_c = m_hbm.at[cid]

        for b in range(NB):
            pltpu.async_copy(m_c.at[srcv.at[b]], bufs[b], sems[b])

        zerov = jnp.zeros((lanes,), dt)

        def zb(i, carry):
            zbuf[i // vpr, pl.ds((i % vpr) * lanes, lanes)] = zerov
            return carry

        lax.fori_loop(0, ZR * vpr, zb, 0)

        for j in range(RPT // ZR):
            pltpu.sync_copy(zbuf, acc.at[pl.ds(sid * RPT + j * ZR, ZR), :])
        plsc.subcore_barrier()

        def chunk(j, carry):
            for b in range(NB):
                q = j * NB + b
                pltpu.make_async_copy(m_c.at[srcv.at[q]], bufs[b], sems[b]).wait()
                pltpu.sync_copy(bufs[b], acc.at[dstv.at[q]], add=True)

                @pl.when(q + NB < NCH2)
                def _():
                    pltpu.async_copy(m_c.at[srcv.at[q + NB]], bufs[b], sems[b])

            return carry

        lax.fori_loop(0, NCH2 // NB, chunk, 0)
        plsc.subcore_barrier()

        pltpu.sync_copy(
            acc.at[pl.ds(sid * RPT, RPT), :],
            out_hbm.at[cid, pl.ds(sid * RPT, RPT), :],
        )

    return agg


def _make_agg(F, dt=jnp.float32, nb=4):
    @functools.partial(
        pl.kernel,
        mesh=_MESH,
        compiler_params=_SC_PARAMS,
        out_type=jax.ShapeDtypeStruct((NC, NP, F), dt),
        scratch_types=(
            [
                pltpu.VMEM((NCH, C), jnp.int32),
                pltpu.VMEM((NCH, C), jnp.int32),
            ]
            + [pltpu.VMEM((C, F), dt) for _ in range(nb)]
            + [pltpu.VMEM((ZR, F), dt)]
            + [pltpu.VMEM_SHARED((NP, F), dt)]
            + [pltpu.SemaphoreType.DMA for _ in range(nb)]
        ),
    )
    def agg(m_hbm, src_hbm, dst_hbm, out_hbm, srcv, dstv, *scr):
        bufs = scr[:nb]
        zbuf = scr[nb]
        acc = scr[nb + 1]
        sems = scr[nb + 2:]
        cid = lax.axis_index("c")
        sid = lax.axis_index("s")
        NB = nb

        pltpu.sync_copy(src_hbm.at[sid, pl.ds(cid * NCH, NCH)], srcv)
        pltpu.sync_copy(dst_hbm.at[sid, pl.ds(cid * NCH, NCH)], dstv)

        for b in range(NB):
            pltpu.async_copy(m_hbm.at[srcv.at[b]], bufs[b], sems[b])

        lanes = 16 if dt == jnp.float32 else 32
        vpr = F // lanes
        zerov = jnp.zeros((lanes,), dt)

        def zb(i, carry):
            zbuf[i // vpr, pl.ds((i % vpr) * lanes, lanes)] = zerov
            return carry

        lax.fori_loop(0, ZR * vpr, zb, 0)

        for j in range(RPT // ZR):
            pltpu.sync_copy(zbuf, acc.at[pl.ds(sid * RPT + j * ZR, ZR), :])
        plsc.subcore_barrier()

        def chunk(j, carry):
            for b in range(NB):
                q = j * NB + b
                pltpu.make_async_copy(m_hbm.at[srcv.at[q]], bufs[b], sems[b]).wait()
                pltpu.sync_copy(bufs[b], acc.at[dstv.at[q]], add=True)

                @pl.when(q + NB < NCH)
                def _():
                    pltpu.async_copy(m_hbm.at[srcv.at[q + NB]], bufs[b], sems[b])

            return carry

        lax.fori_loop(0, NCH // NB, chunk, 0)
        plsc.subcore_barrier()

        pltpu.sync_copy(
            acc.at[pl.ds(sid * RPT, RPT), :],
            out_hbm.at[cid, pl.ds(sid * RPT, RPT), :],
        )

    return agg


_agg128 = _make_agg_fsplit(64, jnp.bfloat16, nb=8)
_agg64 = _make_agg(64, jnp.bfloat16, nb=8)
_agg16 = _make_agg(16, nb=8)


_B = 2000


def _tc0_body(degp_ref, x_ref, m1_ref, no_ref, ni_ref):
    dego = jnp.zeros((N,), jnp.float32)
    degi = jnp.zeros((N,), jnp.float32)
    for w in range(NW):
        dego = dego + degp_ref[pl.ds(w * 2 * N, N)]
        degi = degi + degp_ref[pl.ds(w * 2 * N + N, N)]
    deg = jnp.stack([dego, degi], axis=0)
    norm = jnp.where(deg > 0, lax.rsqrt(deg), 0.0)
    nT = jnp.transpose(norm)
    no = jnp.broadcast_to(nT[:, 0:1], (N, 128))
    ni = jnp.broadcast_to(nT[:, 1:2], (N, 128))
    no_ref[...] = no
    ni_ref[...] = ni
    m1 = (x_ref[...] * no).astype(jnp.bfloat16)
    m1_ref[0] = m1[:, :64]
    m1_ref[1] = m1[:, 64:]


def _tc0(degp, x):
    return pl.pallas_call(
        _tc0_body,
        grid=(1,),
        in_specs=[
            pl.BlockSpec((NW * 2 * N,), lambda i: (0,)),
            pl.BlockSpec((N, 128), lambda i: (0, 0)),
        ],
        out_specs=[
            pl.BlockSpec((2, N, 64), lambda i: (0, 0, 0)),
            pl.BlockSpec((N, 128), lambda i: (0, 0)),
            pl.BlockSpec((N, 128), lambda i: (0, 0)),
        ],
        out_shape=[
            jax.ShapeDtypeStruct((2, N, 64), jnp.bfloat16),
            jax.ShapeDtypeStruct((N, 128), jnp.float32),
            jax.ShapeDtypeStruct((N, 128), jnp.float32),
        ],
    )(degp, x)


def _tc1_body(p_ref, ni_ref, no_ref, W1_ref, b1_ref, W2_ref, m2_ref):
    agg = jnp.concatenate([p_ref[0], p_ref[1]], axis=1).astype(jnp.float32) * ni_ref[...]
    h1 = jnp.maximum(
        jnp.dot(agg, W1_ref[...], preferred_element_type=jnp.float32)
        + b1_ref[...],
        0.0,
    )
    m2_ref[...] = (
        jnp.dot(h1, W2_ref[...], preferred_element_type=jnp.float32)
        * no_ref[...][:, :64]
    ).astype(jnp.bfloat16)


def _tc1(p1, ni, no, W1, b1, W2):
    return pl.pallas_call(
        _tc1_body,
        grid=(N // _B,),
        in_specs=[
            pl.BlockSpec((2, _B, 64), lambda i: (0, i, 0)),
            pl.BlockSpec((_B, 128), lambda i: (i, 0)),
            pl.BlockSpec((_B, 128), lambda i: (i, 0)),
            pl.BlockSpec((128, 128), lambda i: (0, 0)),
            pl.BlockSpec((1, 128), lambda i: (0, 0)),
            pl.BlockSpec((128, 64), lambda i: (0, 0)),
        ],
        out_specs=pl.BlockSpec((_B, 64), lambda i: (i, 0)),
        out_shape=jax.ShapeDtypeStruct((N, 64), jnp.bfloat16),
    )(p1, ni, no, W1, b1, W2)


def _tc2_body(p_ref, ni_ref, no_ref, b2_ref, W3_ref, m3_ref):
    h2 = jnp.maximum(
        (p_ref[0].astype(jnp.float32) + p_ref[1].astype(jnp.float32))
        * ni_ref[...][:, :64]
        + b2_ref[...],
        0.0,
    )
    m3_ref[...] = (
        jnp.dot(h2, W3_ref[...], preferred_element_type=jnp.float32)
        * no_ref[...][:, :16]
    )


def _tc2(p2, ni, no, b2, W3):
    return pl.pallas_call(
        _tc2_body,
        grid=(N // _B,),
        in_specs=[
            pl.BlockSpec((2, _B, 64), lambda i: (0, i, 0)),
            pl.BlockSpec((_B, 128), lambda i: (i, 0)),
            pl.BlockSpec((_B, 128), lambda i: (i, 0)),
            pl.BlockSpec((1, 64), lambda i: (0, 0)),
            pl.BlockSpec((64, 16), lambda i: (0, 0)),
        ],
        out_specs=pl.BlockSpec((_B, 16), lambda i: (i, 0)),
        out_shape=jax.ShapeDtypeStruct((N, 16), jnp.float32),
    )(p2, ni, no, b2, W3)


def _tc3_body(p_ref, ni_ref, b3_ref, h3_ref):
    h3_ref[...] = (p_ref[0] + p_ref[1]) * ni_ref[...][:, :16] + b3_ref[...]


def _tc3(p3, ni, b3):
    return pl.pallas_call(
        _tc3_body,
        grid=(N // _B,),
        in_specs=[
            pl.BlockSpec((2, _B, 16), lambda i: (0, i, 0)),
            pl.BlockSpec((_B, 128), lambda i: (i, 0)),
            pl.BlockSpec((1, 16), lambda i: (0, 0)),
        ],
        out_specs=pl.BlockSpec((_B, 16), lambda i: (i, 0)),
        out_shape=jax.ShapeDtypeStruct((N, 16), jnp.float32),
    )(p3, ni, b3)


def kernel(x, edge_index, W1, b1, W2, b2, W3, b3):
    src = edge_index[0].astype(jnp.int32)
    dst = edge_index[1].astype(jnp.int32)
    src_t = src.reshape(NS, NCH2, C)
    dst_t = dst.reshape(NS, NCH2, C)

    degp = _deg_kernel(src, dst)
    m1, no, ni = _tc0(degp, x)

    p1 = _agg128(m1, src_t, dst_t)
    m2 = _tc1(p1, ni, no, W1, b1.reshape(1, 128), W2)

    p2 = _agg64(m2, src_t, dst_t)
    m3 = _tc2(p2, ni, no, b2.reshape(1, 64), W3)

    p3 = _agg16(m3, src_t, dst_t)
    return _tc3(p3, ni, b3.reshape(1, 16))
